# baseline (speedup 1.0000x reference)
import functools

import numpy as np
import jax
import jax.numpy as jnp
from jax.experimental import pallas as pl
from jax.experimental.pallas import tpu as pltpu

_M = 64
_N = 64
_MN = _M * _N
_DIM = 256
_DECAY = 1000
_SIGMA = 32.0
_SIGMA_OP_SQ = np.float32((_SIGMA * np.exp(-1.0 / _DECAY)) ** 2)


_INV_S2 = np.float32(1.0 / _SIGMA_OP_SQ)


def _som_tile_kernel(x_ref, w_ref, b2_ref, pq_ref, o_ref, *, tb):
    x = x_ref[...]
    w = w_ref[...]
    dn = (((1,), (1,)), ((), ()))
    xw2 = jax.lax.dot_general(x * -2.0, w, dn,
                              preferred_element_type=jnp.float32)
    b2 = b2_ref[...]
    a2 = jnp.sum(x * x, axis=1, keepdims=True)
    d2 = (a2 + b2) + xw2
    m2 = jnp.maximum(jnp.min(d2, axis=1, keepdims=True), 0.0)
    dmin = jnp.sqrt(m2)
    up = jax.lax.bitcast_convert_type(
        jax.lax.bitcast_convert_type(dmin, jnp.uint32) + 1, jnp.float32)
    bound = jnp.maximum(dmin * up, m2)
    col = jax.lax.broadcasted_iota(jnp.int32, (1, _MN), 1)
    idx = jnp.min(jnp.where(d2 <= bound, col, _MN), axis=1, keepdims=True)
    bif = (idx >> 6).astype(jnp.float32)
    bjf = (idx & 63).astype(jnp.float32)
    i64 = jax.lax.broadcasted_iota(jnp.int32, (1, 64), 1).astype(jnp.float32)
    di = i64 - bif
    dj = i64 - bjf
    ai = (di * di) * -_INV_S2
    aj = (dj * dj) * -_INV_S2
    aa = jnp.concatenate([ai, aj], axis=1)
    arg = jax.lax.dot_general(aa, pq_ref[...], (((1,), (0,)), ((), ())),
                              preferred_element_type=jnp.float32)
    o_ref[...] = jnp.exp(arg)


@functools.partial(jax.jit, static_argnames=())
def _som_forward(batch, weights):
    b, dim = batch.shape
    tb = 256
    grid = (b // tb,)
    b2 = jnp.sum(weights * weights, axis=1)[None, :]
    col = np.arange(_MN, dtype=np.int32)
    pq = np.concatenate([
        (col[None, :] >> 6) == np.arange(64, dtype=np.int32)[:, None],
        (col[None, :] & 63) == np.arange(64, dtype=np.int32)[:, None],
    ], axis=0).astype(np.float32)
    return pl.pallas_call(
        functools.partial(_som_tile_kernel, tb=tb),
        grid=grid,
        in_specs=[
            pl.BlockSpec((tb, dim), lambda i: (i, 0)),
            pl.BlockSpec((_MN, dim), lambda i: (0, 0)),
            pl.BlockSpec((1, _MN), lambda i: (0, 0)),
            pl.BlockSpec((128, _MN), lambda i: (0, 0)),
        ],
        out_specs=pl.BlockSpec((tb, _MN), lambda i: (i, 0)),
        out_shape=jax.ShapeDtypeStruct((b, _MN), jnp.float32),
        compiler_params=pltpu.CompilerParams(
            dimension_semantics=("parallel",)),
    )(batch, weights, b2, jnp.asarray(pq))


def kernel(batch, it, weights, locations):
    del it, locations
    return _som_forward(batch, weights)


# speedup vs baseline: 1.0996x; 1.0996x over previous
import functools

import numpy as np
import jax
import jax.numpy as jnp
from jax.experimental import pallas as pl
from jax.experimental.pallas import tpu as pltpu

_M = 64
_N = 64
_MN = _M * _N
_DIM = 256
_DECAY = 1000
_SIGMA = 32.0
_SIGMA_OP_SQ = np.float32((_SIGMA * np.exp(-1.0 / _DECAY)) ** 2)


_INV_S2 = np.float32(1.0 / _SIGMA_OP_SQ)


def _som_tile_kernel(x_ref, w_ref, b2_ref, pq_ref, o_ref, *, tb):
    x = x_ref[...]
    w = w_ref[...]
    dn = (((1,), (1,)), ((), ()))
    xw2 = jax.lax.dot_general(x * -2.0, w, dn,
                              preferred_element_type=jnp.float32)
    b2 = b2_ref[...]
    a2 = jnp.sum(x * x, axis=1, keepdims=True)
    d2 = (a2 + b2) + xw2
    m2 = jnp.maximum(jnp.min(d2, axis=1, keepdims=True), 0.0)
    dmin = jnp.sqrt(m2)
    up = jax.lax.bitcast_convert_type(
        jax.lax.bitcast_convert_type(dmin, jnp.uint32) + 1, jnp.float32)
    bound = jnp.maximum(dmin * up, m2)
    col = jax.lax.broadcasted_iota(jnp.int32, (1, _MN), 1)
    idx = jnp.min(jnp.where(d2 <= bound, col, _MN), axis=1, keepdims=True)
    bif = (idx >> 6).astype(jnp.float32)
    bjf = (idx & 63).astype(jnp.float32)
    i64 = jax.lax.broadcasted_iota(jnp.int32, (1, 64), 1).astype(jnp.float32)
    di = i64 - bif
    dj = i64 - bjf
    ai = (di * di) * -_INV_S2
    aj = (dj * dj) * -_INV_S2
    aa = jnp.concatenate([ai, aj], axis=1)
    arg = jax.lax.dot_general(aa, pq_ref[...], (((1,), (0,)), ((), ())),
                              preferred_element_type=jnp.float32)
    o_ref[...] = jnp.exp(arg)


@functools.partial(jax.jit, static_argnames=())
def _som_forward(batch, weights):
    b, dim = batch.shape
    tb = 512
    grid = (b // tb,)
    b2 = jnp.sum(weights * weights, axis=1)[None, :]
    col = np.arange(_MN, dtype=np.int32)
    pq = np.concatenate([
        (col[None, :] >> 6) == np.arange(64, dtype=np.int32)[:, None],
        (col[None, :] & 63) == np.arange(64, dtype=np.int32)[:, None],
    ], axis=0).astype(np.float32)
    return pl.pallas_call(
        functools.partial(_som_tile_kernel, tb=tb),
        grid=grid,
        in_specs=[
            pl.BlockSpec((tb, dim), lambda i: (i, 0)),
            pl.BlockSpec((_MN, dim), lambda i: (0, 0)),
            pl.BlockSpec((1, _MN), lambda i: (0, 0)),
            pl.BlockSpec((128, _MN), lambda i: (0, 0)),
        ],
        out_specs=pl.BlockSpec((tb, _MN), lambda i: (i, 0)),
        out_shape=jax.ShapeDtypeStruct((b, _MN), jnp.float32),
        compiler_params=pltpu.CompilerParams(
            dimension_semantics=("parallel",)),
    )(batch, weights, b2, jnp.asarray(pq))


def kernel(batch, it, weights, locations):
    del it, locations
    return _som_forward(batch, weights)


# speedup vs baseline: 1.1415x; 1.0381x over previous
import functools

import numpy as np
import jax
import jax.numpy as jnp
from jax.experimental import pallas as pl
from jax.experimental.pallas import tpu as pltpu

_M = 64
_N = 64
_MN = _M * _N
_DIM = 256
_DECAY = 1000
_SIGMA = 32.0
_SIGMA_OP_SQ = np.float32((_SIGMA * np.exp(-1.0 / _DECAY)) ** 2)


_INV_S2 = np.float32(1.0 / _SIGMA_OP_SQ)


def _som_tile_kernel(x_ref, w_ref, b2_ref, pq_ref, o_ref, *, tb):
    x = x_ref[...]
    w = w_ref[...]
    dn = (((1,), (1,)), ((), ()))
    xw2 = jax.lax.dot_general(x * -2.0, w, dn,
                              preferred_element_type=jnp.float32)
    b2 = b2_ref[...]
    a2 = jnp.sum(x * x, axis=1, keepdims=True)
    d2 = (a2 + b2) + xw2
    m2 = jnp.maximum(jnp.min(d2, axis=1, keepdims=True), 0.0)
    dmin = jnp.sqrt(m2)
    up = jax.lax.bitcast_convert_type(
        jax.lax.bitcast_convert_type(dmin, jnp.uint32) + 1, jnp.float32)
    bound = jnp.maximum(dmin * up, m2)
    col = jax.lax.broadcasted_iota(jnp.int32, (1, _MN), 1)
    idx = jnp.min(jnp.where(d2 <= bound, col, _MN), axis=1, keepdims=True)
    bif = (idx >> 6).astype(jnp.float32)
    bjf = (idx & 63).astype(jnp.float32)
    i64 = jax.lax.broadcasted_iota(jnp.int32, (1, 64), 1).astype(jnp.float32)
    di = i64 - bif
    dj = i64 - bjf
    ai = (di * di) * -_INV_S2
    aj = (dj * dj) * -_INV_S2
    aa = jnp.concatenate([ai, aj], axis=1)
    arg = jax.lax.dot_general(aa, pq_ref[...], (((1,), (0,)), ((), ())),
                              preferred_element_type=jnp.float32)
    o_ref[...] = jnp.exp(arg)


@functools.partial(jax.jit, static_argnames=())
def _som_forward(batch, weights):
    b, dim = batch.shape
    tb = 1024
    grid = (b // tb,)
    b2 = jnp.sum(weights * weights, axis=1)[None, :]
    col = np.arange(_MN, dtype=np.int32)
    pq = np.concatenate([
        (col[None, :] >> 6) == np.arange(64, dtype=np.int32)[:, None],
        (col[None, :] & 63) == np.arange(64, dtype=np.int32)[:, None],
    ], axis=0).astype(np.float32)
    return pl.pallas_call(
        functools.partial(_som_tile_kernel, tb=tb),
        grid=grid,
        in_specs=[
            pl.BlockSpec((tb, dim), lambda i: (i, 0)),
            pl.BlockSpec((_MN, dim), lambda i: (0, 0)),
            pl.BlockSpec((1, _MN), lambda i: (0, 0)),
            pl.BlockSpec((128, _MN), lambda i: (0, 0)),
        ],
        out_specs=pl.BlockSpec((tb, _MN), lambda i: (i, 0)),
        out_shape=jax.ShapeDtypeStruct((b, _MN), jnp.float32),
        compiler_params=pltpu.CompilerParams(
            dimension_semantics=("parallel",)),
    )(batch, weights, b2, jnp.asarray(pq))


def kernel(batch, it, weights, locations):
    del it, locations
    return _som_forward(batch, weights)
